# baseline (speedup 1.0000x reference)
import functools

import jax
import jax.numpy as jnp
from jax import lax
from jax.experimental import pallas as pl
from jax.experimental.pallas import tpu as pltpu
from jax.experimental.pallas import tpu_sc as plsc

_N = 10000
_E = 160000
_D = 256
_EPS = 1e-5

_NC = 2
_NS = 16
_LANES = 16

_EDGES_PER_TILE = _E // _NS
_BATCH = 80
_NBATCH = _EDGES_PER_TILE // _BATCH

_DEG_PAD = 10240
_DEG_PER_TILE = _DEG_PAD // _NS

_NW = _NC * _NS
_RANGE = 320
_NPAD = _NW * _RANGE
_SCAN = 4000
_NSCAN = _E // _SCAN
_LISTP = 4048
_DUMMY = _RANGE
_ACC_ROWS = _RANGE + 8
_ROWB = 48
_NSUB = _SCAN // _ROWB + 1


def _sc_degrees(src, dst, zeros_deg):
    mesh = plsc.VectorSubcoreMesh(core_axis_name="c", subcore_axis_name="s")

    @functools.partial(
        pl.kernel,
        mesh=mesh,
        out_type=(
            jax.ShapeDtypeStruct((_DEG_PAD,), jnp.float32),
            jax.ShapeDtypeStruct((_DEG_PAD,), jnp.float32),
        ),
        scratch_types=[
            pltpu.VMEM_SHARED((_DEG_PAD,), jnp.float32),
            pltpu.VMEM((_BATCH,), jnp.int32),
            pltpu.VMEM((_BATCH,), jnp.float32),
        ],
    )
    def kern(src_hbm, dst_hbm, zdeg_hbm, outdeg_hbm, indeg_hbm, sdeg, idx_v, ones_v):
        c = lax.axis_index("c")
        s = lax.axis_index("s")

        off = pl.multiple_of(s * _DEG_PER_TILE, 8)
        pltpu.sync_copy(zdeg_hbm.at[pl.ds(0, _DEG_PER_TILE)], sdeg.at[pl.ds(off, _DEG_PER_TILE)])
        for k in range(_BATCH // _LANES):
            ones_v[pl.ds(k * _LANES, _LANES)] = jnp.full((_LANES,), 1.0, jnp.float32)
        plsc.subcore_barrier()

        def body(j, _):
            base = pl.multiple_of(s * _EDGES_PER_TILE + j * _BATCH, 8)

            @pl.when(c == 0)
            def _():
                pltpu.sync_copy(src_hbm.at[pl.ds(base, _BATCH)], idx_v)

            @pl.when(c == 1)
            def _():
                pltpu.sync_copy(dst_hbm.at[pl.ds(base, _BATCH)], idx_v)

            pltpu.sync_copy(ones_v, sdeg.at[idx_v], add=True)
            return ()

        lax.fori_loop(0, _NBATCH, body, ())
        plsc.subcore_barrier()

        @pl.when(c == 0)
        def _():
            pltpu.sync_copy(sdeg.at[pl.ds(off, _DEG_PER_TILE)], outdeg_hbm.at[pl.ds(off, _DEG_PER_TILE)])

        @pl.when(c == 1)
        def _():
            pltpu.sync_copy(sdeg.at[pl.ds(off, _DEG_PER_TILE)], indeg_hbm.at[pl.ds(off, _DEG_PER_TILE)])

    return kern(src, dst, zeros_deg)


def _sc_aggregate(xs, src, dst, zeros_rows):
    mesh = plsc.VectorSubcoreMesh(core_axis_name="c", subcore_axis_name="s")

    @functools.partial(
        pl.kernel,
        mesh=mesh,
        out_type=jax.ShapeDtypeStruct((_NPAD, _D), jnp.float32),
        compiler_params=pltpu.CompilerParams(needs_layout_passes=False),
        scratch_types=[
            pltpu.VMEM((_ACC_ROWS, _D), jnp.float32),
            pltpu.VMEM((_SCAN,), jnp.int32),
            pltpu.VMEM((_SCAN,), jnp.int32),
            pltpu.VMEM((_LISTP,), jnp.int32),
            pltpu.VMEM((_LISTP,), jnp.int32),
            pltpu.VMEM((_ROWB, _D), jnp.float32),
            pltpu.VMEM((_ROWB, _D), jnp.float32),
            pltpu.VMEM((_RANGE,), jnp.int32),
            pltpu.SemaphoreType.DMA,
            pltpu.SemaphoreType.DMA,
        ],
    )
    def kern(xs_hbm, src_hbm, dst_hbm, zrows_hbm, agg_hbm,
             acc, src_v, dst_v, msrc, mloc, rows_a, rows_b, widx, sem_a, sem_b):
        c = lax.axis_index("c")
        s = lax.axis_index("s")
        wid = s * _NC + c

        pltpu.sync_copy(zrows_hbm, acc)

        def zfill(i, _):
            msrc[pl.ds(i * _LANES, _LANES)] = jnp.zeros((_LANES,), jnp.int32)
            return ()

        lax.fori_loop(0, _LISTP // _LANES, zfill, ())

        def scan_batch(b, _):
            ebase = pl.multiple_of(b * _SCAN, 8)
            pltpu.sync_copy(src_hbm.at[pl.ds(ebase, _SCAN)], src_v)
            pltpu.sync_copy(dst_hbm.at[pl.ds(ebase, _SCAN)], dst_v)

            def compact(i, cnt):
                sl = pl.ds(i * _LANES, _LANES)
                d = dst_v[sl]
                m = (d & (_NW - 1)) == wid
                loc = d >> 5
                pc = plsc.all_reduce_population_count(m)[0].astype(jnp.int32)
                plsc.store_compressed(msrc.at[pl.ds(cnt, _LANES)], src_v[sl], mask=m)
                plsc.store_compressed(mloc.at[pl.ds(cnt, _LANES)], loc, mask=m)
                return cnt + pc

            cnt = lax.fori_loop(0, _SCAN // _LANES, compact, jnp.int32(0))

            for k in range(_ROWB // _LANES):
                mloc[pl.ds(cnt + k * _LANES, _LANES)] = jnp.full((_LANES,), _DUMMY, jnp.int32)

            def accum_from(rows_v, off):
                def accum16(jj, _):
                    co = jj * _LANES
                    loc16 = mloc[pl.ds(off + co, _LANES)]
                    for j in range(_LANES):
                        r = loc16[j]
                        i = co + j
                        for k in range(_D // _LANES):
                            fsl = pl.ds(k * _LANES, _LANES)
                            plsc.addupdate(acc.at[r, fsl], rows_v[i, fsl])
                    return ()

                lax.fori_loop(0, _ROWB // _LANES, accum16, ())

            @pl.when(jnp.int32(0) < cnt)
            def _():
                pltpu.async_copy(xs_hbm.at[msrc.at[pl.ds(0, _ROWB)]], rows_a, sem_a)

            def sub(g, _):
                off = g * _ROWB

                @pl.when(off < cnt)
                def _():
                    osl = pl.ds(pl.multiple_of(off, 8), _ROWB)
                    noff = off + _ROWB
                    nosl = pl.ds(pl.multiple_of(noff, 8), _ROWB)
                    even = (g % 2) == 0

                    @pl.when((noff < cnt) & even)
                    def _():
                        pltpu.async_copy(xs_hbm.at[msrc.at[nosl]], rows_b, sem_b)

                    @pl.when((noff < cnt) & jnp.logical_not(even))
                    def _():
                        pltpu.async_copy(xs_hbm.at[msrc.at[nosl]], rows_a, sem_a)

                    @pl.when(even)
                    def _():
                        pltpu.make_async_copy(xs_hbm.at[msrc.at[osl]], rows_a, sem_a).wait()
                        accum_from(rows_a, off)

                    @pl.when(jnp.logical_not(even))
                    def _():
                        pltpu.make_async_copy(xs_hbm.at[msrc.at[osl]], rows_b, sem_b).wait()
                        accum_from(rows_b, off)

                return ()

            lax.fori_loop(0, _NSUB, sub, ())
            return ()

        lax.fori_loop(0, _NSCAN, scan_batch, ())

        lanes32 = jnp.arange(_LANES, dtype=jnp.int32) * _NW
        for k in range(_RANGE // _LANES):
            widx[pl.ds(k * _LANES, _LANES)] = (wid + k * _LANES * _NW) + lanes32
        pltpu.sync_copy(acc.at[pl.ds(0, _RANGE)], agg_hbm.at[widx])

    return kern(xs, src, dst, zeros_rows)


def _tc_prescale(x, outdeg):
    def body(x_ref, od_ref, xs_ref):
        norm = lax.rsqrt(jnp.maximum(od_ref[...], 1.0))
        xs_ref[...] = x_ref[...] * norm

    return pl.pallas_call(
        body,
        out_shape=jax.ShapeDtypeStruct((_N, _D), jnp.float32),
    )(x, outdeg)


def _tc_layers(agg, indeg, Ws, bs, gs, betas):
    def body(agg_ref, ind_ref, w_ref, b_ref, g_ref, be_ref, out_ref):
        innorm = lax.rsqrt(jnp.maximum(ind_ref[...], 1.0))
        scaled = agg_ref[...] * innorm
        z = jnp.dot(scaled, w_ref[0], preferred_element_type=jnp.float32)
        h = jnp.maximum(z + b_ref[0, 0], 0.0)
        mean = jnp.mean(h, axis=0, keepdims=True)
        var = jnp.mean(jnp.square(h - mean), axis=0, keepdims=True)
        out_ref[...] = g_ref[0, 0] * (h - mean) * lax.rsqrt(var + _EPS) + be_ref[0, 0]

    return pl.pallas_call(
        body,
        grid=(3,),
        in_specs=[
            pl.BlockSpec((_N, _D), lambda i: (0, 0)),
            pl.BlockSpec((_N, 1), lambda i: (0, 0)),
            pl.BlockSpec((1, _D, _D), lambda i: (i, 0, 0)),
            pl.BlockSpec((1, 1, _D), lambda i: (i, 0, 0)),
            pl.BlockSpec((1, 1, _D), lambda i: (i, 0, 0)),
            pl.BlockSpec((1, 1, _D), lambda i: (i, 0, 0)),
        ],
        out_specs=pl.BlockSpec((_N, _D), lambda i: (0, i)),
        out_shape=jax.ShapeDtypeStruct((_N, 3 * _D), jnp.float32),
    )(agg, indeg, Ws, bs, gs, betas)


def kernel(x, edge_index, W0, b0, gamma0, beta0, W1, b1, gamma1, beta1, W2, b2, gamma2, beta2):
    src = edge_index[0]
    dst = edge_index[1]

    zeros_deg = jnp.zeros((_DEG_PER_TILE,), jnp.float32)
    outdeg, indeg = _sc_degrees(src, dst, zeros_deg)
    od = outdeg[:_N].reshape(_N, 1)
    ind = indeg[:_N].reshape(_N, 1)

    xs = _tc_prescale(x, od)

    zeros_rows = jnp.zeros((_ACC_ROWS, _D), jnp.float32)
    agg = _sc_aggregate(xs, src, dst, zeros_rows)[:_N]

    Ws = jnp.stack([W0, W1, W2])
    bs = jnp.stack([b0, b1, b2]).reshape(3, 1, _D)
    gs = jnp.stack([gamma0, gamma1, gamma2]).reshape(3, 1, _D)
    betas = jnp.stack([beta0, beta1, beta2]).reshape(3, 1, _D)
    return _tc_layers(agg, ind, Ws, bs, gs, betas)


# speedup vs baseline: 1.2917x; 1.2917x over previous
import functools

import jax
import jax.numpy as jnp
from jax import lax
from jax.experimental import pallas as pl
from jax.experimental.pallas import tpu as pltpu
from jax.experimental.pallas import tpu_sc as plsc

_N = 10000
_E = 160000
_D = 256
_EPS = 1e-5

_NC = 2
_NS = 16
_LANES = 16

_EDGES_PER_TILE = _E // _NS
_BATCH = 80
_NBATCH = _EDGES_PER_TILE // _BATCH

_DEG_PAD = 10240
_DEG_PER_TILE = _DEG_PAD // _NS

_NW = _NC * _NS
_RANGE = 320
_NPAD = _NW * _RANGE
_SCAN = 4000
_NSCAN = _E // _SCAN
_LISTP = 4048
_DUMMY = _RANGE
_ACC_ROWS = _RANGE + 8
_ROWB = 48
_NSUB = _SCAN // _ROWB + 1


def _sc_degrees(src, dst, zeros_deg):
    mesh = plsc.VectorSubcoreMesh(core_axis_name="c", subcore_axis_name="s")

    @functools.partial(
        pl.kernel,
        mesh=mesh,
        out_type=(
            jax.ShapeDtypeStruct((_DEG_PAD,), jnp.float32),
            jax.ShapeDtypeStruct((_DEG_PAD,), jnp.float32),
        ),
        scratch_types=[
            pltpu.VMEM_SHARED((_DEG_PAD,), jnp.float32),
            pltpu.VMEM((_BATCH,), jnp.int32),
            pltpu.VMEM((_BATCH,), jnp.float32),
        ],
    )
    def kern(src_hbm, dst_hbm, zdeg_hbm, outdeg_hbm, indeg_hbm, sdeg, idx_v, ones_v):
        c = lax.axis_index("c")
        s = lax.axis_index("s")

        off = pl.multiple_of(s * _DEG_PER_TILE, 8)
        pltpu.sync_copy(zdeg_hbm.at[pl.ds(0, _DEG_PER_TILE)], sdeg.at[pl.ds(off, _DEG_PER_TILE)])
        for k in range(_BATCH // _LANES):
            ones_v[pl.ds(k * _LANES, _LANES)] = jnp.full((_LANES,), 1.0, jnp.float32)
        plsc.subcore_barrier()

        def body(j, _):
            base = pl.multiple_of(s * _EDGES_PER_TILE + j * _BATCH, 8)

            @pl.when(c == 0)
            def _():
                pltpu.sync_copy(src_hbm.at[pl.ds(base, _BATCH)], idx_v)

            @pl.when(c == 1)
            def _():
                pltpu.sync_copy(dst_hbm.at[pl.ds(base, _BATCH)], idx_v)

            pltpu.sync_copy(ones_v, sdeg.at[idx_v], add=True)
            return ()

        lax.fori_loop(0, _NBATCH, body, ())
        plsc.subcore_barrier()

        @pl.when(c == 0)
        def _():
            pltpu.sync_copy(sdeg.at[pl.ds(off, _DEG_PER_TILE)], outdeg_hbm.at[pl.ds(off, _DEG_PER_TILE)])

        @pl.when(c == 1)
        def _():
            pltpu.sync_copy(sdeg.at[pl.ds(off, _DEG_PER_TILE)], indeg_hbm.at[pl.ds(off, _DEG_PER_TILE)])

    return kern(src, dst, zeros_deg)


def _sc_aggregate(xs, src, dst, zeros_rows):
    mesh = plsc.VectorSubcoreMesh(core_axis_name="c", subcore_axis_name="s")

    @functools.partial(
        pl.kernel,
        mesh=mesh,
        out_type=jax.ShapeDtypeStruct((_NPAD, _D), jnp.float32),
        compiler_params=pltpu.CompilerParams(needs_layout_passes=False),
        scratch_types=[
            pltpu.VMEM((_ACC_ROWS, _D), jnp.float32),
            pltpu.VMEM((_SCAN,), jnp.int32),
            pltpu.VMEM((_SCAN,), jnp.int32),
            pltpu.VMEM((_LISTP,), jnp.int32),
            pltpu.VMEM((_LISTP,), jnp.int32),
            pltpu.VMEM((_ROWB, _D), jnp.float32),
            pltpu.VMEM((_ROWB, _D), jnp.float32),
            pltpu.VMEM((_RANGE,), jnp.int32),
            pltpu.SemaphoreType.DMA,
            pltpu.SemaphoreType.DMA,
        ],
    )
    def kern(xs_hbm, src_hbm, dst_hbm, zrows_hbm, agg_hbm,
             acc, src_v, dst_v, msrc, mloc, rows_a, rows_b, widx, sem_a, sem_b):
        c = lax.axis_index("c")
        s = lax.axis_index("s")
        wid = s * _NC + c

        pltpu.sync_copy(zrows_hbm, acc)

        def zfill(i, _):
            msrc[pl.ds(i * _LANES, _LANES)] = jnp.zeros((_LANES,), jnp.int32)
            return ()

        lax.fori_loop(0, _LISTP // _LANES, zfill, ())

        def scan_batch(b, _):
            ebase = pl.multiple_of(b * _SCAN, 8)
            pltpu.sync_copy(src_hbm.at[pl.ds(ebase, _SCAN)], src_v)
            pltpu.sync_copy(dst_hbm.at[pl.ds(ebase, _SCAN)], dst_v)

            def compact(i, cnt):
                sl = pl.ds(i * _LANES, _LANES)
                d = dst_v[sl]
                m = (d & (_NW - 1)) == wid
                loc = d >> 5
                pc = plsc.all_reduce_population_count(m)[0].astype(jnp.int32)
                plsc.store_compressed(msrc.at[pl.ds(cnt, _LANES)], src_v[sl], mask=m)
                plsc.store_compressed(mloc.at[pl.ds(cnt, _LANES)], loc, mask=m)
                return cnt + pc

            cnt = lax.fori_loop(0, _SCAN // _LANES, compact, jnp.int32(0))

            for k in range(_ROWB // _LANES):
                mloc[pl.ds(cnt + k * _LANES, _LANES)] = jnp.full((_LANES,), _DUMMY, jnp.int32)

            def accum_from(rows_v, off):
                def accum16(jj, _):
                    co = jj * _LANES
                    loc16 = mloc[pl.ds(off + co, _LANES)]
                    for j in range(_LANES):
                        r = loc16[j]
                        i = co + j
                        vals = [rows_v[i, pl.ds(k * _LANES, _LANES)]
                                for k in range(_D // _LANES)]
                        for k in range(_D // _LANES):
                            fsl = pl.ds(k * _LANES, _LANES)
                            plsc.addupdate(acc.at[r, fsl], vals[k])
                    return ()

                lax.fori_loop(0, _ROWB // _LANES, accum16, ())

            @pl.when(jnp.int32(0) < cnt)
            def _():
                pltpu.async_copy(xs_hbm.at[msrc.at[pl.ds(0, _ROWB)]], rows_a, sem_a)

            def sub(g, _):
                off = g * _ROWB

                @pl.when(off < cnt)
                def _():
                    osl = pl.ds(pl.multiple_of(off, 8), _ROWB)
                    noff = off + _ROWB
                    nosl = pl.ds(pl.multiple_of(noff, 8), _ROWB)
                    even = (g % 2) == 0

                    @pl.when((noff < cnt) & even)
                    def _():
                        pltpu.async_copy(xs_hbm.at[msrc.at[nosl]], rows_b, sem_b)

                    @pl.when((noff < cnt) & jnp.logical_not(even))
                    def _():
                        pltpu.async_copy(xs_hbm.at[msrc.at[nosl]], rows_a, sem_a)

                    @pl.when(even)
                    def _():
                        pltpu.make_async_copy(xs_hbm.at[msrc.at[osl]], rows_a, sem_a).wait()
                        accum_from(rows_a, off)

                    @pl.when(jnp.logical_not(even))
                    def _():
                        pltpu.make_async_copy(xs_hbm.at[msrc.at[osl]], rows_b, sem_b).wait()
                        accum_from(rows_b, off)

                return ()

            lax.fori_loop(0, _NSUB, sub, ())
            return ()

        lax.fori_loop(0, _NSCAN, scan_batch, ())

        lanes32 = jnp.arange(_LANES, dtype=jnp.int32) * _NW
        for k in range(_RANGE // _LANES):
            widx[pl.ds(k * _LANES, _LANES)] = (wid + k * _LANES * _NW) + lanes32
        pltpu.sync_copy(acc.at[pl.ds(0, _RANGE)], agg_hbm.at[widx])

    return kern(xs, src, dst, zeros_rows)


def _tc_prescale(x, outdeg):
    def body(x_ref, od_ref, xs_ref):
        norm = lax.rsqrt(jnp.maximum(od_ref[...], 1.0))
        xs_ref[...] = x_ref[...] * norm

    return pl.pallas_call(
        body,
        out_shape=jax.ShapeDtypeStruct((_N, _D), jnp.float32),
    )(x, outdeg)


def _tc_layers(agg, indeg, Ws, bs, gs, betas):
    def body(agg_ref, ind_ref, w_ref, b_ref, g_ref, be_ref, out_ref):
        innorm = lax.rsqrt(jnp.maximum(ind_ref[...], 1.0))
        scaled = agg_ref[...] * innorm
        z = jnp.dot(scaled, w_ref[0], preferred_element_type=jnp.float32)
        h = jnp.maximum(z + b_ref[0, 0], 0.0)
        mean = jnp.mean(h, axis=0, keepdims=True)
        var = jnp.mean(jnp.square(h - mean), axis=0, keepdims=True)
        out_ref[...] = g_ref[0, 0] * (h - mean) * lax.rsqrt(var + _EPS) + be_ref[0, 0]

    return pl.pallas_call(
        body,
        grid=(3,),
        in_specs=[
            pl.BlockSpec((_N, _D), lambda i: (0, 0)),
            pl.BlockSpec((_N, 1), lambda i: (0, 0)),
            pl.BlockSpec((1, _D, _D), lambda i: (i, 0, 0)),
            pl.BlockSpec((1, 1, _D), lambda i: (i, 0, 0)),
            pl.BlockSpec((1, 1, _D), lambda i: (i, 0, 0)),
            pl.BlockSpec((1, 1, _D), lambda i: (i, 0, 0)),
        ],
        out_specs=pl.BlockSpec((_N, _D), lambda i: (0, i)),
        out_shape=jax.ShapeDtypeStruct((_N, 3 * _D), jnp.float32),
    )(agg, indeg, Ws, bs, gs, betas)


def kernel(x, edge_index, W0, b0, gamma0, beta0, W1, b1, gamma1, beta1, W2, b2, gamma2, beta2):
    src = edge_index[0]
    dst = edge_index[1]

    zeros_deg = jnp.zeros((_DEG_PER_TILE,), jnp.float32)
    outdeg, indeg = _sc_degrees(src, dst, zeros_deg)
    od = outdeg[:_N].reshape(_N, 1)
    ind = indeg[:_N].reshape(_N, 1)

    xs = _tc_prescale(x, od)

    zeros_rows = jnp.zeros((_ACC_ROWS, _D), jnp.float32)
    agg = _sc_aggregate(xs, src, dst, zeros_rows)[:_N]

    Ws = jnp.stack([W0, W1, W2])
    bs = jnp.stack([b0, b1, b2]).reshape(3, 1, _D)
    gs = jnp.stack([gamma0, gamma1, gamma2]).reshape(3, 1, _D)
    betas = jnp.stack([beta0, beta1, beta2]).reshape(3, 1, _D)
    return _tc_layers(agg, ind, Ws, bs, gs, betas)
